# baseline (speedup 1.0000x reference)
import functools

import jax
import jax.numpy as jnp
from jax import lax
from jax.experimental import pallas as pl
from jax.experimental.pallas import tpu as pltpu
from jax.experimental.pallas import tpu_sc as plsc

HIDDEN = 2048
HEAD_DIM = 128
ROTARY_DIM = HEAD_DIM
BASE = 10000.0

NUM_CORES = 2
NUM_SUBCORES = 16
NUM_WORKERS = NUM_CORES * NUM_SUBCORES

CHUNK = 8
NBUF = 4


def _gather_body(n_per_worker, num_chunks, ids_hbm, table_hbm, out_hbm,
                 idx_v, rows, gsem, osem):
    wid = lax.axis_index("s") * NUM_CORES + lax.axis_index("c")
    base = wid * n_per_worker
    per_row = n_per_worker * NUM_WORKERS // ids_hbm.shape[0]
    r = base // per_row
    c = base % per_row
    pltpu.sync_copy(ids_hbm.at[r, pl.ds(c, n_per_worker)], idx_v)

    def start(k, buf):
        pltpu.async_copy(table_hbm.at[idx_v.at[pl.ds(k * CHUNK, CHUNK)]],
                         rows[buf], gsem[buf])

    def wait_g(k, buf):
        pltpu.make_async_copy(table_hbm.at[idx_v.at[pl.ds(k * CHUNK, CHUNK)]],
                              rows[buf], gsem[buf]).wait()

    def start_out(k, buf):
        pltpu.async_copy(rows[buf], out_hbm.at[pl.ds(base + k * CHUNK, CHUNK)],
                         osem[buf])

    def wait_out(k, buf):
        pltpu.make_async_copy(rows[buf],
                              out_hbm.at[pl.ds(base + k * CHUNK, CHUNK)],
                              osem[buf]).wait()

    for b in range(NBUF - 1):
        start(b, b)
    wait_g(0, 0)
    start_out(0, 0)
    start(NBUF - 1, NBUF - 1)

    def group(g, _):
        k0 = 1 + g * NBUF
        for b in range(NBUF):
            k = k0 + b
            buf = (1 + b) % NBUF

            @pl.when(k < num_chunks)
            def _():
                wait_g(k, buf)
                start_out(k, buf)
                nxt = k + NBUF - 1

                @pl.when(nxt < num_chunks)
                def _():
                    wait_out(k - 1, (buf + NBUF - 1) % NBUF)
                    start(nxt, (buf + NBUF - 1) % NBUF)
        return ()

    ngroups = (num_chunks - 1 + NBUF - 1) // NBUF
    lax.fori_loop(0, ngroups, group, (), unroll=False)

    for k in range(num_chunks - NBUF, num_chunks):
        wait_out(k, k % NBUF)


def _sc_gather(ids, table):
    n = ids.shape[0] * ids.shape[1]
    n_per_worker = n // NUM_WORKERS
    num_chunks = n_per_worker // CHUNK
    mesh = plsc.VectorSubcoreMesh(core_axis_name="c", subcore_axis_name="s",
                                  num_cores=NUM_CORES,
                                  num_subcores=NUM_SUBCORES)
    body = functools.partial(_gather_body, n_per_worker, num_chunks)
    return pl.kernel(
        body,
        out_type=jax.ShapeDtypeStruct((n, HIDDEN), table.dtype),
        mesh=mesh,
        scratch_types=[
            pltpu.VMEM((n_per_worker,), jnp.int32),
            [pltpu.VMEM((CHUNK, HIDDEN), jnp.float32)] * NBUF,
            [pltpu.SemaphoreType.DMA] * NBUF,
            [pltpu.SemaphoreType.DMA] * NBUF,
        ],
    )(ids, table)


TBLOCK = 512


def _rope_body(invf_ref, cos_ref, sin_ref, pid_ref):
    t0 = pl.program_id(1) * TBLOCK
    t_idx = t0 + lax.broadcasted_iota(jnp.int32, (1, TBLOCK, ROTARY_DIM), 1)
    pos = t_idx.astype(jnp.float32)
    ang = pos * invf_ref[...][None, :, :]
    cos_ref[...] = jnp.cos(ang)
    sin_ref[...] = jnp.sin(ang)
    pid_ref[...] = lax.broadcasted_iota(jnp.int32, pid_ref.shape, 1)


def _tc_rope(b, t, dtype):
    inv_freq = 1.0 / (BASE ** (jnp.arange(0, ROTARY_DIM, 2,
                                          dtype=jnp.float32) / ROTARY_DIM))
    invf = jnp.concatenate([inv_freq, inv_freq])[None, :]
    grid = (b, t // TBLOCK)
    return pl.pallas_call(
        _rope_body,
        grid=grid,
        in_specs=[pl.BlockSpec((1, ROTARY_DIM), lambda i, j: (0, 0))],
        out_specs=[
            pl.BlockSpec((1, TBLOCK, ROTARY_DIM), lambda i, j: (i, j, 0)),
            pl.BlockSpec((1, TBLOCK, ROTARY_DIM), lambda i, j: (i, j, 0)),
            pl.BlockSpec((b, t), lambda i, j: (0, 0)),
        ],
        out_shape=[
            jax.ShapeDtypeStruct((b, t, ROTARY_DIM), dtype),
            jax.ShapeDtypeStruct((b, t, ROTARY_DIM), dtype),
            jax.ShapeDtypeStruct((b, t), jnp.int32),
        ],
    )(invf)


def kernel(input_ids, attention_mask, table):
    b, t = input_ids.shape
    hidden = _sc_gather(input_ids, table).reshape(b, t, HIDDEN)
    cos, sin, position_ids = _tc_rope(b, t, table.dtype)
    return (hidden, attention_mask, position_ids, cos, sin)
